# baseline (speedup 1.0000x reference)
import functools

import jax
import jax.numpy as jnp
from jax import lax
from jax.experimental import pallas as pl
from jax.experimental.pallas import tpu as pltpu
from jax.experimental.pallas import tpu_sc as plsc

D_MODEL = 128
H = 30
W = 30
NCOLORS = 11
P = H * W
B = 1024
NC, NS = 2, 16
NW = NC * NS
PLCH = 4
SLOT = 32
NBUF = 3
NPART = 4
PART = B // NPART


def _fused_body(color_ref, row_ref, col_ref, out_ref):
    out_ref[...] = (color_ref[...][:, None, None, :]
                    + row_ref[...][None, :, None, :]
                    + col_ref[...][None, None, :, :])


def _build_fused(color_table, row_table, col_table):
    out = pl.pallas_call(
        _fused_body,
        out_shape=jax.ShapeDtypeStruct((NCOLORS, H, W, D_MODEL), jnp.float32),
    )(color_table, row_table, col_table)
    return out.reshape(NCOLORS * P, D_MODEL)


_mesh = plsc.VectorSubcoreMesh(core_axis_name="c", subcore_axis_name="s",
                               num_cores=NC, num_subcores=NS)


def _make_sc_gather(nbatch):
    bpw = nbatch // NW
    ppw = bpw * H
    nch = ppw // PLCH
    cpw = ppw * W
    rpw = ppw * SLOT
    assert nch % NBUF == 0

    @functools.partial(
        pl.kernel,
        out_type=jax.ShapeDtypeStruct((nbatch * H * SLOT, D_MODEL),
                                      jnp.float32),
        mesh=_mesh,
        scratch_types=[
            pltpu.VMEM((cpw + 16,), jnp.int32),
            pltpu.VMEM((nch, PLCH * SLOT), jnp.int32),
            pltpu.VMEM((NBUF, PLCH * SLOT, D_MODEL), jnp.float32),
            [pltpu.SemaphoreType.DMA] * NBUF,
            [pltpu.SemaphoreType.DMA] * NBUF,
        ],
    )
    def _sc_gather(fused_hbm, grid_hbm, out_hbm, grid_v, idx_v, rows_v,
                   gsems, ssems):
        wid = lax.axis_index("s") * NC + lax.axis_index("c")
        rbase = wid * rpw

        pltpu.sync_copy(grid_hbm.at[pl.ds(wid * cpw, cpw)],
                        grid_v.at[pl.ds(0, cpw)])

        iota = lax.iota(jnp.int32, 16)

        def idx_body(c, h0):
            for k in range(PLCH):
                hk = h0 + k
                hk = jnp.where(hk >= H, hk - H, hk)
                f = c * (PLCH * W) + k * W
                pb = hk * W + iota
                idx_v[c, pl.ds(k * SLOT, 16)] = (
                    grid_v[pl.ds(f, 16)] * P + pb)
                idx_v[c, pl.ds(k * SLOT + 16, 16)] = jnp.where(
                    iota < W - 16,
                    grid_v[pl.ds(f + 16, 16)] * P + pb + 16, 0)
            h1 = h0 + PLCH
            return jnp.where(h1 >= H, h1 - H, h1)

        lax.fori_loop(0, nch, idx_body, jnp.int32(0))

        def g_desc(c, b):
            return pltpu.make_async_copy(
                fused_hbm.at[idx_v.at[c]], rows_v.at[b], gsems[b])

        def s_desc(c, b):
            return pltpu.make_async_copy(
                rows_v.at[b],
                out_hbm.at[pl.ds(rbase + c * (PLCH * SLOT), PLCH * SLOT)],
                ssems[b])

        g_desc(0, 0).start()
        for c in range(NBUF):
            b = c % NBUF
            g_desc(c, b).wait()
            s_desc(c, b).start()
            nb = (b + 1) % NBUF
            if c == NBUF - 1:
                s_desc(c + 1 - NBUF, nb).wait()
            g_desc(c + 1, nb).start()

        def outer(t, _):
            for b in range(NBUF):
                c = t * NBUF + b
                g_desc(c, b).wait()
                s_desc(c, b).start()
                nb = (b + 1) % NBUF
                s_desc(c + 1 - NBUF, nb).wait()
                g_desc(c + 1, nb).start()
            return 0

        lax.fori_loop(1, nch // NBUF - 1, outer, 0)

        for c in range(nch - NBUF, nch):
            b = c % NBUF
            g_desc(c, b).wait()
            s_desc(c, b).start()
            if c + 1 < nch:
                nb = (b + 1) % NBUF
                s_desc(c + 1 - NBUF, nb).wait()
                g_desc(c + 1, nb).start()
        for c in range(nch - NBUF, nch):
            s_desc(c, c % NBUF).wait()

    return _sc_gather


_sc_gather_part = _make_sc_gather(PART)


def _relayout_body(slab_ref, out_ref):
    for h in range(H):
        out_ref[0, h] = slab_ref[pl.ds(h * SLOT, W)]


def _relayout_chain_body(slab_ref, prev_ref, out_ref):
    del prev_ref
    _relayout_body(slab_ref, out_ref)


def _relayout_first(slab):
    return pl.pallas_call(
        _relayout_body,
        grid=(PART,),
        in_specs=[pl.BlockSpec((H * SLOT, D_MODEL), lambda b: (b, 0))],
        out_specs=pl.BlockSpec((1, H, W, D_MODEL), lambda b: (b, 0, 0, 0)),
        out_shape=jax.ShapeDtypeStruct((B, H, W, D_MODEL), jnp.float32),
    )(slab)


def _make_relayout_chain(part_idx):
    base = part_idx * PART
    return pl.pallas_call(
        _relayout_chain_body,
        grid=(PART,),
        in_specs=[
            pl.BlockSpec((H * SLOT, D_MODEL), lambda b: (b, 0)),
            pl.BlockSpec(memory_space=pl.ANY),
        ],
        out_specs=pl.BlockSpec((1, H, W, D_MODEL),
                               lambda b: (b + base, 0, 0, 0)),
        out_shape=jax.ShapeDtypeStruct((B, H, W, D_MODEL), jnp.float32),
        input_output_aliases={1: 0},
    )


_relayout_chains = [_make_relayout_chain(i) for i in range(1, NPART)]


def kernel(grid, color_table, row_table, col_table):
    fused = _build_fused(color_table, row_table, col_table)
    gflat = grid.reshape(B * P)
    slabs = [
        _sc_gather_part(fused, gflat[i * PART * P:(i + 1) * PART * P])
        for i in range(NPART)
    ]
    out = _relayout_first(slabs[0])
    for i in range(1, NPART):
        out = _relayout_chains[i - 1](slabs[i], out)
    return out


# speedup vs baseline: 1.1448x; 1.1448x over previous
import functools

import jax
import jax.numpy as jnp
from jax import lax
from jax.experimental import pallas as pl
from jax.experimental.pallas import tpu as pltpu
from jax.experimental.pallas import tpu_sc as plsc

D_MODEL = 128
H = 30
W = 30
NCOLORS = 11
P = H * W
B = 1024
NPL = B * H
NC, NS = 2, 16
NW = NC * NS
PPW = NPL // NW
BPW = B // NW
PLCH = 4
NCH = PPW // PLCH
CPW = PPW * W
NBUF = 3


def _fused_body(color_ref, row_ref, col_ref, out_ref):
    out_ref[...] = (color_ref[...][:, None, None, :]
                    + row_ref[...][None, :, None, :]
                    + col_ref[...][None, None, :, :])


def _build_fused(color_table, row_table, col_table):
    out = pl.pallas_call(
        _fused_body,
        out_shape=jax.ShapeDtypeStruct((NCOLORS, H, W, D_MODEL), jnp.float32),
    )(color_table, row_table, col_table)
    return out.reshape(NCOLORS * P, D_MODEL)


_mesh = plsc.VectorSubcoreMesh(core_axis_name="c", subcore_axis_name="s",
                               num_cores=NC, num_subcores=NS)


@functools.partial(
    pl.kernel,
    out_type=jax.ShapeDtypeStruct((B, H, W, D_MODEL), jnp.float32),
    mesh=_mesh,
    compiler_params=pltpu.CompilerParams(use_tc_tiling_on_sc=True),
    scratch_types=[
        pltpu.VMEM((CPW + 16,), jnp.int32),
        pltpu.VMEM((NCH, PLCH * 32), jnp.int32),
        pltpu.VMEM((NBUF, PLCH * 32, D_MODEL), jnp.float32),
        [pltpu.SemaphoreType.DMA] * NBUF,
        [pltpu.SemaphoreType.DMA] * NBUF,
    ],
)
def _sc_gather(fused_hbm, grid_hbm, out_hbm, grid_v, idx_v, rows_v,
               gsems, ssems):
    wid = lax.axis_index("s") * NC + lax.axis_index("c")
    bbase = wid * BPW

    pltpu.sync_copy(grid_hbm.at[pl.ds(wid * CPW, CPW)],
                    grid_v.at[pl.ds(0, CPW)])

    iota = lax.iota(jnp.int32, 16)

    def idx_body(c, h0):
        for k in range(PLCH):
            hk = h0 + k
            hk = jnp.where(hk >= H, hk - H, hk)
            f = c * (PLCH * W) + k * W
            pb = hk * W + iota
            idx_v[c, pl.ds(k * 32, 16)] = grid_v[pl.ds(f, 16)] * P + pb
            idx_v[c, pl.ds(k * 32 + 16, 16)] = jnp.where(
                iota < W - 16,
                grid_v[pl.ds(f + 16, 16)] * P + pb + 16, 0)
        h1 = h0 + PLCH
        return jnp.where(h1 >= H, h1 - H, h1)

    lax.fori_loop(0, NCH, idx_body, jnp.int32(0))

    def g_desc(c, b):
        return pltpu.make_async_copy(
            fused_hbm.at[idx_v.at[c]], rows_v.at[b], gsems[b])

    def start_gather(c, b):
        g_desc(c, b).start()

    def wait_gather(c, b):
        g_desc(c, b).wait()

    def s_start(b, bloc, h0):
        for k in range(PLCH):
            hk = h0 + k
            wr = (hk >= H).astype(jnp.int32)
            hk2 = hk - wr * H
            pltpu.make_async_copy(
                rows_v.at[b, pl.ds(k * 32, W)],
                out_hbm.at[bbase + bloc + wr, hk2], ssems[b]).start()

    def s_start_static(c, b):
        h0 = (c * PLCH) % H
        assert h0 + PLCH <= H
        for k in range(PLCH):
            pltpu.make_async_copy(
                rows_v.at[b, pl.ds(k * 32, W)],
                out_hbm.at[bbase + (c * PLCH) // H, h0 + k],
                ssems[b]).start()

    def s_wait(b):
        for k in range(PLCH):
            pltpu.make_async_copy(
                rows_v.at[b, pl.ds(k * 32, W)], out_hbm.at[0, 0],
                ssems[b]).wait()

    start_gather(0, 0)
    for c in range(NBUF):
        b = c % NBUF
        wait_gather(c, b)
        s_start_static(c, b)
        nb = (b + 1) % NBUF
        if c == NBUF - 1:
            s_wait(nb)
        start_gather(c + 1, nb)

    def outer(t, state):
        bloc, h0 = state
        for b in range(NBUF):
            c = t * NBUF + b
            wait_gather(c, b)
            s_start(b, bloc, h0)
            nb = (b + 1) % NBUF
            s_wait(nb)
            start_gather(c + 1, nb)
            h1 = h0 + PLCH
            wrap = h1 >= H
            h0 = jnp.where(wrap, h1 - H, h1)
            bloc = bloc + wrap.astype(jnp.int32)
        return bloc, h0

    c0 = NBUF
    lax.fori_loop(1, NCH // NBUF - 1, outer,
                  (jnp.int32((c0 * PLCH) // H), jnp.int32((c0 * PLCH) % H)))

    for c in range(NCH - NBUF, NCH):
        b = c % NBUF
        wait_gather(c, b)
        s_start_static(c, b)
        if c + 1 < NCH:
            nb = (b + 1) % NBUF
            s_wait(nb)
            start_gather(c + 1, nb)
    for c in range(NCH - NBUF, NCH):
        s_wait(c % NBUF)


def kernel(grid, color_table, row_table, col_table):
    fused = _build_fused(color_table, row_table, col_table)
    return _sc_gather(fused, grid.reshape(B * P))


# speedup vs baseline: 4.9389x; 4.3144x over previous
import functools

import jax
import jax.numpy as jnp
from jax import lax
from jax.experimental import pallas as pl
from jax.experimental.pallas import tpu as pltpu
from jax.experimental.pallas import tpu_sc as plsc

D_MODEL = 128
H = 30
W = 30
NCOLORS = 11
P = H * W
B = 1024
NPL = B * H
NC, NS = 2, 16
NW = NC * NS
PPW = NPL // NW
BPW = B // NW
PLCH = 4
NCH = PPW // PLCH
CPW = PPW * W
NBUF = 3


def _fused_body(color_ref, row_ref, col_ref, out_ref):
    out_ref[...] = (color_ref[...][:, None, None, :]
                    + row_ref[...][None, :, None, :]
                    + col_ref[...][None, None, :, :])


def _build_fused(color_table, row_table, col_table):
    out = pl.pallas_call(
        _fused_body,
        out_shape=jax.ShapeDtypeStruct((NCOLORS, H, W, D_MODEL), jnp.float32),
    )(color_table, row_table, col_table)
    return out.reshape(NCOLORS * P, D_MODEL)


_mesh = plsc.VectorSubcoreMesh(core_axis_name="c", subcore_axis_name="s",
                               num_cores=NC, num_subcores=NS)


@functools.partial(
    pl.kernel,
    out_type=jax.ShapeDtypeStruct((B, H, W, D_MODEL), jnp.float32),
    mesh=_mesh,
    compiler_params=pltpu.CompilerParams(use_tc_tiling_on_sc=True),
    scratch_types=[
        pltpu.VMEM((CPW + 16,), jnp.int32),
        pltpu.VMEM((NCH, PLCH * 32), jnp.int32),
        pltpu.VMEM((NBUF, PLCH, W, D_MODEL), jnp.float32),
        [pltpu.SemaphoreType.DMA] * NBUF,
        [pltpu.SemaphoreType.DMA] * NBUF,
    ],
)
def _sc_gather(fused_hbm, grid_hbm, out_hbm, grid_v, idx_v, rows_v,
               gsems, ssems):
    wid = lax.axis_index("s") * NC + lax.axis_index("c")
    bbase = wid * BPW

    pltpu.sync_copy(grid_hbm.at[pl.ds(wid * CPW, CPW)],
                    grid_v.at[pl.ds(0, CPW)])

    iota = lax.iota(jnp.int32, 16)

    def idx_body(c, h0):
        for k in range(PLCH):
            hk = h0 + k
            hk = jnp.where(hk >= H, hk - H, hk)
            f = c * (PLCH * W) + k * W
            pb = hk * W + iota
            idx_v[c, pl.ds(k * 32, 16)] = grid_v[pl.ds(f, 16)] * P + pb
            idx_v[c, pl.ds(k * 32 + 16, 16)] = (
                grid_v[pl.ds(f + 16, 16)] * P + pb + 16)
        h1 = h0 + PLCH
        return jnp.where(h1 >= H, h1 - H, h1)

    lax.fori_loop(0, NCH, idx_body, jnp.int32(0))

    def g_descs(c, b):
        return [pltpu.make_async_copy(
                    fused_hbm.at[idx_v.at[c, pl.ds(k * 32, W)]],
                    rows_v.at[b, k], gsems[b])
                for k in range(PLCH)]

    def start_gather(c, b):
        for d in g_descs(c, b):
            d.start()

    def wait_gather(c, b):
        for d in g_descs(c, b):
            d.wait()

    def s_start(b, bloc, h0):
        bg = bbase + bloc

        @pl.when(h0 != H - 2)
        def _():
            pltpu.make_async_copy(
                rows_v.at[b], out_hbm.at[bg, pl.ds(h0, PLCH)],
                ssems[b]).start()

        @pl.when(h0 == H - 2)
        def _():
            pltpu.make_async_copy(
                rows_v.at[b, pl.ds(0, 2)],
                out_hbm.at[bg, pl.ds(H - 2, 2)], ssems[b]).start()
            pltpu.make_async_copy(
                rows_v.at[b, pl.ds(2, 2)],
                out_hbm.at[bg + 1, pl.ds(0, 2)], ssems[b]).start()

    def s_start_static(c, b):
        h0 = (c * PLCH) % H
        assert h0 != H - 2
        pltpu.make_async_copy(
            rows_v.at[b],
            out_hbm.at[bbase + (c * PLCH) // H, pl.ds(h0, PLCH)],
            ssems[b]).start()

    def s_wait(b):
        pltpu.make_async_copy(
            rows_v.at[b], out_hbm.at[0, pl.ds(0, PLCH)], ssems[b]).wait()

    start_gather(0, 0)
    for c in range(NBUF):
        b = c % NBUF
        wait_gather(c, b)
        s_start_static(c, b)
        nb = (b + 1) % NBUF
        if c == NBUF - 1:
            s_wait(nb)
        start_gather(c + 1, nb)

    def outer(t, state):
        bloc, h0 = state
        for b in range(NBUF):
            c = t * NBUF + b
            wait_gather(c, b)
            s_start(b, bloc, h0)
            nb = (b + 1) % NBUF
            s_wait(nb)
            start_gather(c + 1, nb)
            h1 = h0 + PLCH
            wrap = h1 >= H
            h0 = jnp.where(wrap, h1 - H, h1)
            bloc = bloc + wrap.astype(jnp.int32)
        return bloc, h0

    c0 = NBUF
    lax.fori_loop(1, NCH // NBUF - 1, outer,
                  (jnp.int32((c0 * PLCH) // H), jnp.int32((c0 * PLCH) % H)))

    for c in range(NCH - NBUF, NCH):
        b = c % NBUF
        wait_gather(c, b)
        s_start_static(c, b)
        if c + 1 < NCH:
            nb = (b + 1) % NBUF
            s_wait(nb)
            start_gather(c + 1, nb)
    for c in range(NCH - NBUF, NCH):
        s_wait(c % NBUF)


def kernel(grid, color_table, row_table, col_table):
    fused = _build_fused(color_table, row_table, col_table)
    return _sc_gather(fused, grid.reshape(B * P))
